# baseline (speedup 1.0000x reference)
import jax
import jax.numpy as jnp
from jax.experimental import pallas as pl
from jax.experimental.pallas import tpu as pltpu

N = 64
D = 128
HEADS = 4
DH = 32
L = 2
IB = 16


def _ln(x, s, b):
    mu = x.mean(-1, keepdims=True)
    var = ((x - mu) ** 2).mean(-1, keepdims=True)
    return (x - mu) / jnp.sqrt(var + 1e-05) * s + b


def _adj_kernel(A0_ref, maskf_ref, P_ref, Q_ref, alpha_ref, A_ref, bias_ref):
    S = jnp.dot(P_ref[...], Q_ref[...].T, preferred_element_type=jnp.float32)
    sp = jnp.maximum(S, 0.0) + jnp.log1p(jnp.exp(-jnp.abs(S)))
    A0 = A0_ref[...]
    A = A0 * (1.0 + alpha_ref[0, 0] * sp * maskf_ref[...])
    A_ref[...] = A / (A.sum(-1, keepdims=True) + 1e-08)
    bias_ref[...] = jnp.log(A0 + 1e-08)


def _enc_kernel(X_ref, A_ref, bias_ref, maskf_ref, ident_ref, WpT_ref, bp_ref,
                ln1s_ref, ln1b_ref, qWT_ref, valWT_ref, cvecT_ref, outWT_ref,
                g1w_ref, g1b_ref, g2w_ref, g2b_ref, ln2s_ref, ln2b_ref,
                m1wT_ref, m1b_ref, m2wT_ref, m2b_ref,
                Z_out_ref, S_out_ref):
    x = X_ref[...].reshape(IB * N, D)
    Z = jnp.dot(x, WpT_ref[...], preferred_element_type=jnp.float32) + bp_ref[0]
    A = A_ref[...]
    bias = bias_ref[...]
    maskf = maskf_ref[...]
    bias2 = jnp.concatenate([bias, bias], axis=1)
    mask2 = jnp.concatenate([maskf, maskf], axis=1) > 0
    lself = jax.lax.broadcasted_iota(jnp.int32, (1, 2 * N), 1) < N
    ident = ident_ref[...]
    neg = jnp.float32(-1e30)
    for l in range(L):
        H = _ln(Z, ln1s_ref[l], ln1b_ref[l])
        Xy = jnp.dot(H, qWT_ref[l], preferred_element_type=jnp.float32)
        Xv = jnp.dot(H, valWT_ref[l], preferred_element_type=jnp.float32)
        XyT = jax.lax.dot_general(Xy, ident, (((0,), (0,)), ((), ())),
                                  preferred_element_type=jnp.float32)
        Xy16 = Xy.astype(jnp.bfloat16)
        XyT16 = XyT.astype(jnp.bfloat16)
        cvec = cvecT_ref[:, l:l + 1].astype(jnp.bfloat16)
        mix_rows = []
        y_rows = [None] * IB
        for pb in range(IB // 2):
            b0, b1 = 2 * pb, 2 * pb + 1
            y0 = Xy16[b0 * N:(b0 + 1) * N]
            y1 = Xy16[b1 * N:(b1 + 1) * N]
            QT2 = XyT16[:, b0 * N:b0 * N + 2 * N]
            Qsel = jnp.concatenate(
                [jnp.broadcast_to(y0[:, :, None], (N, D, N)),
                 jnp.broadcast_to(y1[:, :, None], (N, D, N))], axis=2)
            t = Qsel + QT2[None, :, :]
            w = jnp.abs(t) * cvec[None, :, :]
            heads = ([], [])
            for h in range(HEADS):
                hs = h * DH
                e = w[:, hs:hs + DH, :].sum(axis=1).astype(jnp.float32)
                Ssel = jnp.where(
                    lself,
                    Xy[b0 * N:(b0 + 1) * N, hs:hs + DH].sum(-1, keepdims=True),
                    Xy[b1 * N:(b1 + 1) * N, hs:hs + DH].sum(-1, keepdims=True))
                Slane = XyT[hs:hs + DH,
                            b0 * N:b0 * N + 2 * N].sum(axis=0, keepdims=True)
                e = e + 0.6 * (Ssel + Slane) + bias2
                e = jnp.where(mask2, e, neg)
                for half, b in ((0, b0), (1, b1)):
                    eh = e[:, half * N:(half + 1) * N]
                    m = eh.max(-1, keepdims=True)
                    p = jnp.exp(eh - m)
                    attn = p / p.sum(-1, keepdims=True)
                    Vh = Xv[b * N:(b + 1) * N, hs:hs + DH]
                    heads[half].append(jnp.dot(attn, Vh,
                                               preferred_element_type=jnp.float32))
            y_rows[b0] = jnp.concatenate(heads[0], axis=-1)
            y_rows[b1] = jnp.concatenate(heads[1], axis=-1)
        for b in range(IB):
            Hb = H[b * N:(b + 1) * N]
            mix_rows.append(jnp.dot(A, Hb, preferred_element_type=jnp.float32))
        Hmix = jnp.concatenate(mix_rows, axis=0)
        Y = jnp.concatenate(y_rows, axis=0)
        Hattn = jnp.dot(Y, outWT_ref[l], preferred_element_type=jnp.float32)
        U = Z + Hmix + Hattn
        s = U.mean(-1, keepdims=True)
        gp = s * g1w_ref[l][None, :] + g1b_ref[l][None, :]
        gm = gp * jax.nn.sigmoid(gp)
        gs = (gm * g2w_ref[l][None, :]).sum(-1, keepdims=True) + g2b_ref[l, 0]
        U = U * jax.nn.sigmoid(gs)
        V = _ln(U, ln2s_ref[l], ln2b_ref[l])
        V = jnp.dot(V, m1wT_ref[l], preferred_element_type=jnp.float32) + m1b_ref[l]
        V = V * jax.nn.sigmoid(V)
        V = jnp.dot(V, m2wT_ref[l], preferred_element_type=jnp.float32) + m2b_ref[l]
        Z = U + V
    Z3 = Z.reshape(IB, N, D)
    Z_out_ref[...] = Z3
    S_out_ref[...] = Z3.mean(axis=1)


def kernel(X, A0, mask, Wp, bp, P, Q, alpha, ln1_s, ln1_b, linW, valW, attA,
           outW, g1w, g1b, g2w, g2b, ln2_s, ln2_b, m1w, m1b, m2w, m2b):
    B, T, n, d = X.shape
    BT = B * T
    maskf = mask.astype(jnp.float32)

    A, bias = pl.pallas_call(
        _adj_kernel,
        out_shape=(jax.ShapeDtypeStruct((N, N), jnp.float32),
                   jax.ShapeDtypeStruct((N, N), jnp.float32)),
    )(A0, maskf, P, Q, jnp.reshape(alpha, (1, 1)))

    X2 = X.reshape(BT, n, d)
    grid = BT // IB

    aflat = attA.reshape(L, D)
    qWT = jnp.transpose(linW, (0, 2, 1)) * aflat[:, None, :]
    cvecT = (0.4 * jnp.sign(aflat)).T

    def xmap(i):
        return (i, 0, 0)

    def wmap2(i):
        return (0, 0)

    def wmap3(i):
        return (0, 0, 0)

    full2 = lambda shape: pl.BlockSpec(shape, wmap2)
    full3 = lambda shape: pl.BlockSpec(shape, wmap3)

    in_specs = [
        pl.BlockSpec((IB, N, D), xmap),
        full2((N, N)),
        full2((N, N)),
        full2((N, N)),
        full2((IB * N, IB * N)),
        full2((D, D)),
        full2((1, D)),
        full2((L, D)),
        full2((L, D)),
        full3((L, D, D)),
        full3((L, D, D)),
        full2((D, L)),
        full3((L, D, D)),
        full2((L, D)),
        full2((L, D)),
        full2((L, D)),
        full2((L, 1)),
        full2((L, D)),
        full2((L, D)),
        full3((L, D, 4 * D)),
        full2((L, 4 * D)),
        full3((L, 4 * D, D)),
        full2((L, D)),
    ]
    out_specs = (
        pl.BlockSpec((IB, N, D), xmap),
        pl.BlockSpec((IB, D), lambda i: (i, 0)),
    )

    Zf, Sf = pl.pallas_call(
        _enc_kernel,
        grid=(grid,),
        in_specs=in_specs,
        out_specs=out_specs,
        out_shape=(jax.ShapeDtypeStruct((BT, N, D), jnp.float32),
                   jax.ShapeDtypeStruct((BT, D), jnp.float32)),
        compiler_params=pltpu.CompilerParams(
            dimension_semantics=("parallel",)),
    )(
        X2, A, bias, maskf, jnp.eye(IB * N, dtype=jnp.float32),
        Wp.T, bp.reshape(1, D),
        ln1_s, ln1_b,
        qWT, jnp.transpose(valW, (0, 2, 1)),
        cvecT,
        jnp.transpose(outW, (0, 2, 1)),
        g1w.reshape(L, D), g1b, g2w.reshape(L, D), g2b.reshape(L, 1),
        ln2_s, ln2_b,
        jnp.transpose(m1w, (0, 2, 1)), m1b,
        jnp.transpose(m2w, (0, 2, 1)), m2b,
    )
    return Zf.reshape(B, T, n, d), Sf.reshape(B, T, d), A


# speedup vs baseline: 1.3772x; 1.3772x over previous
import jax
import jax.numpy as jnp
from jax.experimental import pallas as pl
from jax.experimental.pallas import tpu as pltpu

N = 64
D = 128
HEADS = 4
DH = 32
L = 2
IB = 8
CH = 8


def _ln(x, s, b):
    mu = x.mean(-1, keepdims=True)
    var = ((x - mu) ** 2).mean(-1, keepdims=True)
    return (x - mu) / jnp.sqrt(var + 1e-05) * s + b


def _adj_kernel(A0_ref, maskf_ref, P_ref, Q_ref, alpha_ref, A_ref, bias_ref):
    S = jnp.dot(P_ref[...], Q_ref[...].T, preferred_element_type=jnp.float32)
    sp = jnp.maximum(S, 0.0) + jnp.log1p(jnp.exp(-jnp.abs(S)))
    A0 = A0_ref[...]
    A = A0 * (1.0 + alpha_ref[0, 0] * sp * maskf_ref[...])
    A_ref[...] = A / (A.sum(-1, keepdims=True) + 1e-08)
    bias_ref[...] = jnp.log(A0 + 1e-08)


def _enc_kernel(X_ref, A_ref, bias_ref, maskf_ref, ident_ref, WpT_ref, bp_ref,
                ln1s_ref, ln1b_ref, qWT_ref, valWT_ref, cvecT_ref, outWT_ref,
                g1w_ref, g1b_ref, g2w_ref, g2b_ref, ln2s_ref, ln2b_ref,
                m1wT_ref, m1b_ref, m2wT_ref, m2b_ref,
                Z_out_ref, S_out_ref):
    x = X_ref[...].reshape(IB * N, D)
    Z = jnp.dot(x, WpT_ref[...], preferred_element_type=jnp.float32) + bp_ref[0]
    A = A_ref[...]
    bias = bias_ref[...]
    maskf = maskf_ref[...]
    bias2 = jnp.concatenate([bias, bias], axis=1)
    mask2 = jnp.concatenate([maskf, maskf], axis=1) > 0
    lself = jax.lax.broadcasted_iota(jnp.int32, (1, 2 * N), 1) < N
    ident = ident_ref[...]
    neg = jnp.float32(-1e30)
    for l in range(L):
        H = _ln(Z, ln1s_ref[l], ln1b_ref[l])
        Xy = jnp.dot(H, qWT_ref[l], preferred_element_type=jnp.float32)
        Xv = jnp.dot(H, valWT_ref[l], preferred_element_type=jnp.float32)
        XyT = jax.lax.dot_general(Xy, ident, (((0,), (0,)), ((), ())),
                                  preferred_element_type=jnp.float32)
        Xy16 = Xy.astype(jnp.bfloat16)
        XyT16 = XyT.astype(jnp.bfloat16)
        cvec = cvecT_ref[:, l:l + 1].astype(jnp.bfloat16)
        Srow = [Xy[:, h * DH:(h + 1) * DH].sum(-1, keepdims=True)
                for h in range(HEADS)]
        SlaneL = [XyT[h * DH:(h + 1) * DH, :].sum(axis=0, keepdims=True)
                  for h in range(HEADS)]
        mix_rows = []
        y_rows = [None] * IB
        for pb in range(IB // 2):
            b0, b1 = 2 * pb, 2 * pb + 1
            y0 = Xy16[b0 * N:(b0 + 1) * N]
            y1 = Xy16[b1 * N:(b1 + 1) * N]
            QT2b = XyT16[None, :, b0 * N:b0 * N + 2 * N]
            e_parts = [[] for _ in range(HEADS)]
            for i0 in range(0, N, CH):
                Qsel = jnp.concatenate(
                    [jnp.broadcast_to(y0[i0:i0 + CH, :, None], (CH, D, N)),
                     jnp.broadcast_to(y1[i0:i0 + CH, :, None], (CH, D, N))],
                    axis=2)
                w = jnp.abs(Qsel + QT2b) * cvec[None, :, :]
                for h in range(HEADS):
                    hs = h * DH
                    a = w[:, hs:hs + 16, :] + w[:, hs + 16:hs + DH, :]
                    a = a[:, 0:8, :] + a[:, 8:16, :]
                    a = a[:, 0:4, :] + a[:, 4:8, :]
                    a = a[:, 0:2, :] + a[:, 2:4, :]
                    e_parts[h].append(a[:, 0, :].astype(jnp.float32)
                                      + a[:, 1, :].astype(jnp.float32))
            heads = ([], [])
            for h in range(HEADS):
                e = jnp.concatenate(e_parts[h], axis=0)
                Ssel = jnp.where(lself,
                                 Srow[h][b0 * N:(b0 + 1) * N],
                                 Srow[h][b1 * N:(b1 + 1) * N])
                e = e + Ssel + SlaneL[h][:, b0 * N:b0 * N + 2 * N] + bias2
                e = jnp.where(mask2, e, neg)
                hs = h * DH
                for half, b in ((0, b0), (1, b1)):
                    eh = e[:, half * N:(half + 1) * N]
                    m = eh.max(-1, keepdims=True)
                    p = jnp.exp(eh - m)
                    attn = p / p.sum(-1, keepdims=True)
                    Vh = Xv[b * N:(b + 1) * N, hs:hs + DH]
                    heads[half].append(jnp.dot(attn, Vh,
                                               preferred_element_type=jnp.float32))
            y_rows[b0] = jnp.concatenate(heads[0], axis=-1)
            y_rows[b1] = jnp.concatenate(heads[1], axis=-1)
        for b in range(IB):
            Hb = H[b * N:(b + 1) * N]
            mix_rows.append(jnp.dot(A, Hb, preferred_element_type=jnp.float32))
        Hmix = jnp.concatenate(mix_rows, axis=0)
        Y = jnp.concatenate(y_rows, axis=0)
        Hattn = jnp.dot(Y, outWT_ref[l], preferred_element_type=jnp.float32)
        U = Z + Hmix + Hattn
        s = U.mean(-1, keepdims=True)
        gp = s * g1w_ref[l][None, :] + g1b_ref[l][None, :]
        gm = gp * jax.nn.sigmoid(gp)
        gs = (gm * g2w_ref[l][None, :]).sum(-1, keepdims=True) + g2b_ref[l, 0]
        U = U * jax.nn.sigmoid(gs)
        V = _ln(U, ln2s_ref[l], ln2b_ref[l])
        V = jnp.dot(V, m1wT_ref[l], preferred_element_type=jnp.float32) + m1b_ref[l]
        V = V * jax.nn.sigmoid(V)
        V = jnp.dot(V, m2wT_ref[l], preferred_element_type=jnp.float32) + m2b_ref[l]
        Z = U + V
    Z3 = Z.reshape(IB, N, D)
    Z_out_ref[...] = Z3
    S_out_ref[...] = Z3.mean(axis=1)


def kernel(X, A0, mask, Wp, bp, P, Q, alpha, ln1_s, ln1_b, linW, valW, attA,
           outW, g1w, g1b, g2w, g2b, ln2_s, ln2_b, m1w, m1b, m2w, m2b):
    B, T, n, d = X.shape
    BT = B * T
    maskf = mask.astype(jnp.float32)

    A, bias = pl.pallas_call(
        _adj_kernel,
        out_shape=(jax.ShapeDtypeStruct((N, N), jnp.float32),
                   jax.ShapeDtypeStruct((N, N), jnp.float32)),
    )(A0, maskf, P, Q, jnp.reshape(alpha, (1, 1)))

    X2 = X.reshape(BT, n, d)
    grid = BT // IB

    aflat = attA.reshape(L, D)
    qWT = jnp.transpose(linW, (0, 2, 1)) * (0.6 * aflat[:, None, :])
    cvecT = ((0.4 / 0.6) * jnp.sign(aflat)).T

    def xmap(i):
        return (i, 0, 0)

    def wmap2(i):
        return (0, 0)

    def wmap3(i):
        return (0, 0, 0)

    full2 = lambda shape: pl.BlockSpec(shape, wmap2)
    full3 = lambda shape: pl.BlockSpec(shape, wmap3)

    in_specs = [
        pl.BlockSpec((IB, N, D), xmap),
        full2((N, N)),
        full2((N, N)),
        full2((N, N)),
        full2((IB * N, IB * N)),
        full2((D, D)),
        full2((1, D)),
        full2((L, D)),
        full2((L, D)),
        full3((L, D, D)),
        full3((L, D, D)),
        full2((D, L)),
        full3((L, D, D)),
        full2((L, D)),
        full2((L, D)),
        full2((L, D)),
        full2((L, 1)),
        full2((L, D)),
        full2((L, D)),
        full3((L, D, 4 * D)),
        full2((L, 4 * D)),
        full3((L, 4 * D, D)),
        full2((L, D)),
    ]
    out_specs = (
        pl.BlockSpec((IB, N, D), xmap),
        pl.BlockSpec((IB, D), lambda i: (i, 0)),
    )

    Zf, Sf = pl.pallas_call(
        _enc_kernel,
        grid=(grid,),
        in_specs=in_specs,
        out_specs=out_specs,
        out_shape=(jax.ShapeDtypeStruct((BT, N, D), jnp.float32),
                   jax.ShapeDtypeStruct((BT, D), jnp.float32)),
        compiler_params=pltpu.CompilerParams(
            dimension_semantics=("parallel",)),
    )(
        X2, A, bias, maskf, jnp.eye(IB * N, dtype=jnp.float32),
        Wp.T, bp.reshape(1, D),
        ln1_s, ln1_b,
        qWT, jnp.transpose(valW, (0, 2, 1)),
        cvecT,
        jnp.transpose(outW, (0, 2, 1)),
        g1w.reshape(L, D), g1b, g2w.reshape(L, D), g2b.reshape(L, 1),
        ln2_s, ln2_b,
        jnp.transpose(m1w, (0, 2, 1)), m1b,
        jnp.transpose(m2w, (0, 2, 1)), m2b,
    )
    return Zf.reshape(B, T, n, d), Sf.reshape(B, T, d), A
